# baseline (speedup 1.0000x reference)
import functools

import jax
import jax.numpy as jnp
from jax import lax
from jax.experimental import pallas as pl
from jax.experimental.pallas import tpu as pltpu
from jax.experimental.pallas import tpu_sc as plsc

N_FEAT = 26
BATCH = 16384
ROWS = 5
DIM = 3
OUT_D = N_FEAT * DIM
OUT_PAD = 128
NC, NS, LANES = 2, 16, 16
NW = NC * NS
B_TILE = BATCH // NW
NVEC = B_TILE // LANES
TBL_WORDS = N_FEAT * ROWS * DIM
REP_WORDS = TBL_WORDS * LANES
T_STRIDE = N_FEAT + 7
G2 = N_FEAT - LANES
ROW_UNROLL = 12

_mesh = plsc.VectorSubcoreMesh(
    core_axis_name="c", subcore_axis_name="s", num_cores=NC, num_subcores=NS
)


@functools.partial(
    pl.kernel,
    out_type=jax.ShapeDtypeStruct((BATCH * OUT_PAD,), jnp.float32),
    mesh=_mesh,
    scratch_types=[
        pltpu.VMEM((N_FEAT, B_TILE), jnp.int32),
        pltpu.VMEM((B_TILE * T_STRIDE,), jnp.int32),
        pltpu.VMEM((REP_WORDS,), jnp.float32),
        pltpu.VMEM((B_TILE * OUT_PAD,), jnp.float32),
        pltpu.SemaphoreType.DMA,
    ],
    compiler_params=pltpu.CompilerParams(needs_layout_passes=False),
)
def _embed_sc(*refs):
    idx_hbm = refs[:N_FEAT]
    tbl_hbm = refs[N_FEAT]
    out_hbm = refs[N_FEAT + 1]
    idx_v, idx_t, tbl_v, out_v, sem = refs[N_FEAT + 2:]

    wid = lax.axis_index("s") * NC + lax.axis_index("c")
    base = wid * B_TILE

    with jax.named_scope("stage_in"):
        copies = [
            pltpu.async_copy(idx_hbm[i].at[pl.ds(base, B_TILE)], idx_v.at[i], sem)
            for i in range(N_FEAT)
        ]
        pltpu.sync_copy(tbl_hbm, tbl_v)
        for c in copies:
            c.wait()

    lane = lax.broadcasted_iota(jnp.int32, (LANES,), 0)
    lane_t = lane * T_STRIDE
    lane3 = lane * DIM
    zeros = jnp.zeros((LANES,), jnp.int32)

    @plsc.parallel_loop(0, B_TILE * T_STRIDE // LANES, unroll=4)
    def _(k):
        idx_t[pl.ds(k * LANES, LANES)] = zeros

    with jax.named_scope("transpose"):

        @plsc.parallel_loop(0, NVEC, unroll=2)
        def _(j):
            rows_t = lane_t + j * (LANES * T_STRIDE)
            for i in range(N_FEAT):
                idx16 = idx_v[i, pl.ds(j * LANES, LANES)]
                addr = idx16 * (DIM * LANES) + (
                    ROWS * DIM * LANES * i + (i % LANES)
                )
                plsc.store_scatter(idx_t, [rows_t + i], addr)

    with jax.named_scope("gather_loop"):

        @plsc.parallel_loop(0, B_TILE, unroll=ROW_UNROLL)
        def _(b):
            tb = b * T_STRIDE
            ob = b * OUT_PAD
            a1 = idx_t[pl.ds(tb, LANES)]
            a2 = idx_t[pl.ds(tb + LANES, LANES)]
            for d in range(DIM):
                v1 = plsc.load_gather(tbl_v, [a1 + d * LANES])
                plsc.store_scatter(out_v, [lane3 + (ob + d)], v1)
                v2 = plsc.load_gather(tbl_v, [a2 + d * LANES])
                plsc.store_scatter(
                    out_v, [lane3 + (ob + LANES * DIM + d)], v2
                )

    with jax.named_scope("store_out"):
        pltpu.sync_copy(
            out_v, out_hbm.at[pl.ds(base * OUT_PAD, B_TILE * OUT_PAD)]
        )


def kernel(f0, f1, f2, f3, f4, f5, f6, f7, f8, f9, f10, f11, f12, f13, f14,
           f15, f16, f17, f18, f19, f20, f21, f22, f23, f24, f25,
           W_f0, W_f1, W_f2, W_f3, W_f4, W_f5, W_f6, W_f7, W_f8, W_f9,
           W_f10, W_f11, W_f12, W_f13, W_f14, W_f15, W_f16, W_f17, W_f18,
           W_f19, W_f20, W_f21, W_f22, W_f23, W_f24, W_f25):
    fs = (f0, f1, f2, f3, f4, f5, f6, f7, f8, f9, f10, f11, f12, f13, f14,
          f15, f16, f17, f18, f19, f20, f21, f22, f23, f24, f25)
    Ws = (W_f0, W_f1, W_f2, W_f3, W_f4, W_f5, W_f6, W_f7, W_f8, W_f9,
          W_f10, W_f11, W_f12, W_f13, W_f14, W_f15, W_f16, W_f17, W_f18,
          W_f19, W_f20, W_f21, W_f22, W_f23, W_f24, W_f25)
    idx = [jnp.asarray(f, jnp.int32) for f in fs]
    tbl = jnp.concatenate([w.astype(jnp.float32) for w in Ws], axis=0)
    rep = jnp.broadcast_to(tbl.reshape(-1)[:, None], (TBL_WORDS, LANES))
    out_flat = _embed_sc(*idx, rep.reshape(-1))
    return out_flat.reshape(BATCH, OUT_PAD)[:, :OUT_D]


# speedup vs baseline: 1.1323x; 1.1323x over previous
import functools

import jax
import jax.numpy as jnp
from jax import lax
from jax.experimental import pallas as pl
from jax.experimental.pallas import tpu as pltpu
from jax.experimental.pallas import tpu_sc as plsc

N_FEAT = 26
BATCH = 16384
ROWS = 5
DIM = 3
OUT_D = N_FEAT * DIM
OUT_PAD = 128
NC, NS, LANES = 2, 16, 16
NW = NC * NS
B_TILE = BATCH // NW
NVEC = B_TILE // LANES
TBL_WORDS = N_FEAT * ROWS * DIM
REP_WORDS = TBL_WORDS * LANES
T_STRIDE = N_FEAT + 7
G2 = N_FEAT - LANES
ROW_UNROLL = 8

_mesh = plsc.VectorSubcoreMesh(
    core_axis_name="c", subcore_axis_name="s", num_cores=NC, num_subcores=NS
)


@functools.partial(
    pl.kernel,
    out_type=jax.ShapeDtypeStruct((BATCH * OUT_PAD,), jnp.float32),
    mesh=_mesh,
    scratch_types=[
        pltpu.VMEM((N_FEAT, B_TILE), jnp.int32),
        pltpu.VMEM((B_TILE * T_STRIDE,), jnp.int32),
        pltpu.VMEM((REP_WORDS,), jnp.float32),
        pltpu.VMEM((B_TILE * OUT_PAD,), jnp.float32),
        pltpu.SemaphoreType.DMA,
        pltpu.SemaphoreType.DMA,
        pltpu.SemaphoreType.DMA,
    ],
    compiler_params=pltpu.CompilerParams(needs_layout_passes=False),
)
def _embed_sc(*refs):
    idx_hbm = refs[:N_FEAT]
    tbl_hbm = refs[N_FEAT]
    out_hbm = refs[N_FEAT + 1]
    idx_v, idx_t, tbl_v, out_v, sem_a, sem_b, sem_t = refs[N_FEAT + 2:]

    wid = lax.axis_index("s") * NC + lax.axis_index("c")
    base = wid * B_TILE

    H = N_FEAT // 2
    with jax.named_scope("stage_in"):
        copies = [
            pltpu.async_copy(
                idx_hbm[i].at[pl.ds(base, B_TILE)], idx_v.at[i],
                sem_a if i < H else sem_b,
            )
            for i in range(N_FEAT)
        ]
        tbl_copy = pltpu.async_copy(tbl_hbm, tbl_v, sem_t)

    lane = lax.broadcasted_iota(jnp.int32, (LANES,), 0)
    lane_t = lane * T_STRIDE
    lane3 = lane * DIM
    zeros = jnp.zeros((LANES,), jnp.int32)

    @plsc.parallel_loop(0, B_TILE * T_STRIDE // LANES, unroll=4)
    def _(k):
        idx_t[pl.ds(k * LANES, LANES)] = zeros

    def _transpose_half(lo, hi):
        @plsc.parallel_loop(0, NVEC, unroll=2)
        def _(j):
            rows_t = lane_t + j * (LANES * T_STRIDE)
            for i in range(lo, hi):
                idx16 = idx_v[i, pl.ds(j * LANES, LANES)]
                addr = idx16 * (DIM * LANES) + (
                    ROWS * DIM * LANES * i + (i % LANES)
                )
                plsc.store_scatter(idx_t, [rows_t + i], addr)

    with jax.named_scope("transpose"):
        for c in copies[:H]:
            c.wait()
        _transpose_half(0, H)
        for c in copies[H:]:
            c.wait()
        _transpose_half(H, N_FEAT)
        tbl_copy.wait()

    with jax.named_scope("gather_loop"):

        @plsc.parallel_loop(0, B_TILE, unroll=ROW_UNROLL)
        def _(b):
            tb = b * T_STRIDE
            ob = b * OUT_PAD
            a1 = idx_t[pl.ds(tb, LANES)]
            a2 = idx_t[pl.ds(tb + LANES, LANES)]
            for d in range(DIM):
                v1 = plsc.load_gather(tbl_v, [a1 + d * LANES])
                plsc.store_scatter(out_v, [lane3 + (ob + d)], v1)
                v2 = plsc.load_gather(tbl_v, [a2 + d * LANES])
                plsc.store_scatter(
                    out_v, [lane3 + (ob + LANES * DIM + d)], v2
                )

    with jax.named_scope("store_out"):
        pltpu.sync_copy(
            out_v, out_hbm.at[pl.ds(base * OUT_PAD, B_TILE * OUT_PAD)]
        )


def kernel(f0, f1, f2, f3, f4, f5, f6, f7, f8, f9, f10, f11, f12, f13, f14,
           f15, f16, f17, f18, f19, f20, f21, f22, f23, f24, f25,
           W_f0, W_f1, W_f2, W_f3, W_f4, W_f5, W_f6, W_f7, W_f8, W_f9,
           W_f10, W_f11, W_f12, W_f13, W_f14, W_f15, W_f16, W_f17, W_f18,
           W_f19, W_f20, W_f21, W_f22, W_f23, W_f24, W_f25):
    fs = (f0, f1, f2, f3, f4, f5, f6, f7, f8, f9, f10, f11, f12, f13, f14,
          f15, f16, f17, f18, f19, f20, f21, f22, f23, f24, f25)
    Ws = (W_f0, W_f1, W_f2, W_f3, W_f4, W_f5, W_f6, W_f7, W_f8, W_f9,
          W_f10, W_f11, W_f12, W_f13, W_f14, W_f15, W_f16, W_f17, W_f18,
          W_f19, W_f20, W_f21, W_f22, W_f23, W_f24, W_f25)
    idx = [jnp.asarray(f, jnp.int32) for f in fs]
    tbl = jnp.concatenate([w.astype(jnp.float32) for w in Ws], axis=0)
    rep = jnp.broadcast_to(tbl.reshape(-1)[:, None], (TBL_WORDS, LANES))
    out_flat = _embed_sc(*idx, rep.reshape(-1))
    return out_flat.reshape(BATCH, OUT_PAD)[:, :OUT_D]


# speedup vs baseline: 1.1932x; 1.0538x over previous
import functools

import jax
import jax.numpy as jnp
from jax import lax
from jax.experimental import pallas as pl
from jax.experimental.pallas import tpu as pltpu
from jax.experimental.pallas import tpu_sc as plsc

N_FEAT = 26
BATCH = 16384
ROWS = 5
DIM = 3
OUT_D = N_FEAT * DIM
OUT_PAD = 128
NC, NS, LANES = 2, 16, 16
NW = NC * NS
B_TILE = BATCH // NW
NVEC = B_TILE // LANES
TBL_WORDS = N_FEAT * ROWS * DIM
REP_WORDS = TBL_WORDS * LANES
T_STRIDE = N_FEAT + 7
G2 = N_FEAT - LANES
ROW_UNROLL = 8

_mesh = plsc.VectorSubcoreMesh(
    core_axis_name="c", subcore_axis_name="s", num_cores=NC, num_subcores=NS
)


@functools.partial(
    pl.kernel,
    out_type=jax.ShapeDtypeStruct((BATCH * OUT_PAD,), jnp.float32),
    mesh=_mesh,
    scratch_types=[
        pltpu.VMEM((N_FEAT, B_TILE), jnp.int32),
        pltpu.VMEM((B_TILE * T_STRIDE,), jnp.int32),
        pltpu.VMEM((REP_WORDS,), jnp.float32),
        pltpu.VMEM((B_TILE * OUT_PAD,), jnp.float32),
        pltpu.SemaphoreType.DMA,
        pltpu.SemaphoreType.DMA,
        pltpu.SemaphoreType.DMA,
        pltpu.SemaphoreType.DMA,
        pltpu.SemaphoreType.DMA,
        pltpu.SemaphoreType.DMA,
    ],
    compiler_params=pltpu.CompilerParams(needs_layout_passes=False),
)
def _embed_sc(*refs):
    idx_hbm = refs[:N_FEAT]
    tbl_hbm = refs[N_FEAT]
    out_hbm = refs[N_FEAT + 1]
    (idx_v, idx_t, tbl_v, out_v,
     sem_a, sem_b, sem_c, sem_d, sem_t, sem_o) = refs[N_FEAT + 2:]

    wid = lax.axis_index("s") * NC + lax.axis_index("c")
    base = wid * B_TILE

    GRP = (0, 7, 13, 20, N_FEAT)
    sems = (sem_a, sem_b, sem_c, sem_d)
    with jax.named_scope("stage_in"):
        copies = [
            pltpu.async_copy(
                idx_hbm[i].at[pl.ds(base, B_TILE)], idx_v.at[i],
                sems[sum(1 for g in GRP[1:4] if i >= g)],
            )
            for i in range(N_FEAT)
        ]
        tbl_copy = pltpu.async_copy(tbl_hbm, tbl_v, sem_t)

    lane = lax.broadcasted_iota(jnp.int32, (LANES,), 0)
    lane_t = lane * T_STRIDE
    lane3 = lane * DIM
    zeros = jnp.zeros((LANES,), jnp.int32)

    @plsc.parallel_loop(0, B_TILE * T_STRIDE // LANES, unroll=4)
    def _(k):
        idx_t[pl.ds(k * LANES, LANES)] = zeros

    def _transpose_half(lo, hi):
        @plsc.parallel_loop(0, NVEC, unroll=2)
        def _(j):
            rows_t = lane_t + j * (LANES * T_STRIDE)
            for i in range(lo, hi):
                idx16 = idx_v[i, pl.ds(j * LANES, LANES)]
                addr = idx16 * (DIM * LANES) + (
                    ROWS * DIM * LANES * i + (i % LANES)
                )
                plsc.store_scatter(idx_t, [rows_t + i], addr)

    with jax.named_scope("transpose"):
        for g in range(4):
            lo, hi = GRP[g], GRP[g + 1]
            for c in copies[lo:hi]:
                c.wait()
            _transpose_half(lo, hi)
        tbl_copy.wait()

    def _gather_rows(lo, hi):
        @plsc.parallel_loop(lo, hi, unroll=ROW_UNROLL)
        def _(b):
            tb = b * T_STRIDE
            ob = b * OUT_PAD
            a1 = idx_t[pl.ds(tb, LANES)]
            a2 = idx_t[pl.ds(tb + LANES, LANES)]
            for d in range(DIM):
                v1 = plsc.load_gather(tbl_v, [a1 + d * LANES])
                plsc.store_scatter(out_v, [lane3 + (ob + d)], v1)
                v2 = plsc.load_gather(tbl_v, [a2 + d * LANES])
                plsc.store_scatter(
                    out_v, [lane3 + (ob + LANES * DIM + d)], v2
                )

    HALF = B_TILE // 2 * OUT_PAD
    with jax.named_scope("gather_loop"):
        _gather_rows(0, B_TILE // 2)
        st1 = pltpu.async_copy(
            out_v.at[pl.ds(0, HALF)],
            out_hbm.at[pl.ds(base * OUT_PAD, HALF)],
            sem_o,
        )
        _gather_rows(B_TILE // 2, B_TILE)

    with jax.named_scope("store_out"):
        pltpu.sync_copy(
            out_v.at[pl.ds(HALF, HALF)],
            out_hbm.at[pl.ds(base * OUT_PAD + HALF, HALF)],
        )
        st1.wait()


def kernel(f0, f1, f2, f3, f4, f5, f6, f7, f8, f9, f10, f11, f12, f13, f14,
           f15, f16, f17, f18, f19, f20, f21, f22, f23, f24, f25,
           W_f0, W_f1, W_f2, W_f3, W_f4, W_f5, W_f6, W_f7, W_f8, W_f9,
           W_f10, W_f11, W_f12, W_f13, W_f14, W_f15, W_f16, W_f17, W_f18,
           W_f19, W_f20, W_f21, W_f22, W_f23, W_f24, W_f25):
    fs = (f0, f1, f2, f3, f4, f5, f6, f7, f8, f9, f10, f11, f12, f13, f14,
          f15, f16, f17, f18, f19, f20, f21, f22, f23, f24, f25)
    Ws = (W_f0, W_f1, W_f2, W_f3, W_f4, W_f5, W_f6, W_f7, W_f8, W_f9,
          W_f10, W_f11, W_f12, W_f13, W_f14, W_f15, W_f16, W_f17, W_f18,
          W_f19, W_f20, W_f21, W_f22, W_f23, W_f24, W_f25)
    idx = [jnp.asarray(f, jnp.int32) for f in fs]
    tbl = jnp.concatenate([w.astype(jnp.float32) for w in Ws], axis=0)
    rep = jnp.broadcast_to(tbl.reshape(-1)[:, None], (TBL_WORDS, LANES))
    out_flat = _embed_sc(*idx, rep.reshape(-1))
    return out_flat.reshape(BATCH, OUT_PAD)[:, :OUT_D]
